# baseline (speedup 1.0000x reference)
import jax
import jax.numpy as jnp
from jax import lax
from jax.experimental import pallas as pl
from jax.experimental.pallas import tpu as pltpu
from jax.experimental.pallas import tpu_sc as plsc

_B = 4096
_L = 200
_D = 64
_ASP = 14
_LANES = 128

_NC = 2
_NS = 16
_NW = _NC * _NS
_RPW = _B // _NW
_VREGS = _D // 16
_LA = 128
_LB = _L - _LA
_OFF = _LANES - _LB


def _pool_body(xa_hbm, xb_hbm, table_hbm, out_hbm, idxa_v, idxb_v,
               bufa0, bufb0, bufa1, bufb1, pooled_v, sem0, sem1):
    c = lax.axis_index("c")
    s = lax.axis_index("s")
    wid = s * _NC + c
    base = wid * _RPW
    pltpu.sync_copy(xa_hbm.at[pl.ds(base, _RPW)], idxa_v)
    pltpu.sync_copy(xb_hbm.at[pl.ds(base, _RPW)], idxb_v)

    def start(r, bufa, bufb, sem):
        pltpu.async_copy(table_hbm.at[idxa_v.at[r]], bufa, sem)
        pltpu.async_copy(
            table_hbm.at[idxb_v.at[r, pl.ds(_OFF, _LB)]], bufb, sem)

    def wait(r, bufa, bufb, sem):
        pltpu.make_async_copy(table_hbm.at[idxa_v.at[r]], bufa, sem).wait()
        pltpu.make_async_copy(
            table_hbm.at[idxb_v.at[r, pl.ds(_OFF, _LB)]], bufb, sem).wait()

    def rowsum(bufa, bufb, r):
        def jbody_a(j, accs):
            out = []
            for u in range(2):
                for k in range(_VREGS):
                    out.append(accs[u * _VREGS + k]
                               + bufa[2 * j + u, pl.ds(k * 16, 16)])
            return tuple(out)

        def jbody_b(j, accs):
            out = []
            for u in range(2):
                for k in range(_VREGS):
                    out.append(accs[u * _VREGS + k]
                               + bufb[2 * j + u, pl.ds(k * 16, 16)])
            return tuple(out)

        accs = tuple(jnp.zeros((16,), jnp.float32) for _ in range(2 * _VREGS))
        accs = lax.fori_loop(0, _LA // 2, jbody_a, accs)
        accs = lax.fori_loop(0, _LB // 2, jbody_b, accs)
        for k in range(_VREGS):
            pooled_v[r, pl.ds(k * 16, 16)] = accs[k] + accs[_VREGS + k]

    start(0, bufa0, bufb0, sem0)

    def pair_body(i, carry):
        r0 = 2 * i
        r1 = r0 + 1
        start(r1, bufa1, bufb1, sem1)
        wait(r0, bufa0, bufb0, sem0)
        rowsum(bufa0, bufb0, r0)

        @pl.when(i < _RPW // 2 - 1)
        def _():
            start(r0 + 2, bufa0, bufb0, sem0)

        wait(r1, bufa1, bufb1, sem1)
        rowsum(bufa1, bufb1, r1)
        return carry

    lax.fori_loop(0, _RPW // 2, pair_body, 0)
    pltpu.sync_copy(pooled_v, out_hbm.at[pl.ds(base, _RPW)])


@jax.jit
def _pool(xa, xb, table):
    mesh = plsc.VectorSubcoreMesh(core_axis_name="c", subcore_axis_name="s",
                                  num_cores=_NC)
    return pl.kernel(
        _pool_body,
        mesh=mesh,
        compiler_params=pltpu.CompilerParams(use_tc_tiling_on_sc=False),
        out_type=jax.ShapeDtypeStruct((_B, _D), jnp.float32),
        scratch_types=[
            pltpu.VMEM((_RPW, _LANES), jnp.int32),
            pltpu.VMEM((_RPW, _LANES), jnp.int32),
            pltpu.VMEM((_LA, _D), jnp.float32),
            pltpu.VMEM((_LB, _D), jnp.float32),
            pltpu.VMEM((_LA, _D), jnp.float32),
            pltpu.VMEM((_LB, _D), jnp.float32),
            pltpu.VMEM((_RPW, _D), jnp.float32),
            pltpu.SemaphoreType.DMA,
            pltpu.SemaphoreType.DMA,
        ],
    )(xa, xb, table)


def _head_body(pooled_ref, len_ref, w_ref, b_ref, o_ref):
    p = pooled_ref[...] / len_ref[...]
    logits = jnp.dot(p, w_ref[...], preferred_element_type=jnp.float32)
    logits = logits + b_ref[...]
    m = jnp.max(logits, axis=-1, keepdims=True)
    e = jnp.exp(logits - m)
    o_ref[...] = e / jnp.sum(e, axis=-1, keepdims=True)


@jax.jit
def _head(pooled, lens, w_pad, b_pad):
    return pl.pallas_call(
        _head_body,
        out_shape=jax.ShapeDtypeStruct((_B, _LANES), jnp.float32),
    )(pooled, lens, w_pad, b_pad)


def kernel(x, x_len, table, W, b):
    xi = x.astype(jnp.int32)
    xa = xi[:, :_LANES]
    xb = xi[:, _L - _LANES:_L]
    pooled = _pool(xa, xb, table)
    lens = x_len.astype(jnp.float32).reshape(_B, 1)
    w_pad = jnp.pad(W, ((0, 0), (0, _LANES - _ASP)))
    b_pad = jnp.concatenate(
        [b, jnp.full((_LANES - _ASP,), -1e30, jnp.float32)]).reshape(1, _LANES)
    out = _head(pooled, lens, w_pad, b_pad)
    return out[:, :_ASP]


# speedup vs baseline: 1.0029x; 1.0029x over previous
import jax
import jax.numpy as jnp
from jax import lax
from jax.experimental import pallas as pl
from jax.experimental.pallas import tpu as pltpu
from jax.experimental.pallas import tpu_sc as plsc

_B = 4096
_L = 200
_D = 64
_ASP = 14
_LANES = 128

_NC = 2
_NS = 16
_NW = _NC * _NS
_RPW = _B // _NW
_VREGS = _D // 16
_LA = 128
_LB = _L - _LA
_OFF = _LANES - _LB


def _pool_body(xa_hbm, xb_hbm, table_hbm, out_hbm, idxa_v, idxb_v,
               bufa0, bufb0, bufa1, bufb1, pooled_v, sem0, sem1):
    c = lax.axis_index("c")
    s = lax.axis_index("s")
    wid = s * _NC + c
    base = wid * _RPW
    pltpu.sync_copy(xa_hbm.at[pl.ds(base, _RPW)], idxa_v)
    pltpu.sync_copy(xb_hbm.at[pl.ds(base, _RPW)], idxb_v)

    def start(r, bufa, bufb, sem):
        pltpu.async_copy(table_hbm.at[idxa_v.at[r]], bufa, sem)
        pltpu.async_copy(
            table_hbm.at[idxb_v.at[r, pl.ds(_OFF, _LB)]], bufb, sem)

    def wait(r, bufa, bufb, sem):
        pltpu.make_async_copy(table_hbm.at[idxa_v.at[r]], bufa, sem).wait()
        pltpu.make_async_copy(
            table_hbm.at[idxb_v.at[r, pl.ds(_OFF, _LB)]], bufb, sem).wait()

    def rowsum(bufa, bufb, r):
        def jbody_a(j, accs):
            out = []
            for u in range(2):
                for k in range(_VREGS):
                    out.append(accs[u * _VREGS + k]
                               + bufa[2 * j + u, pl.ds(k * 16, 16)])
            return tuple(out)

        def jbody_b(j, accs):
            out = []
            for u in range(2):
                for k in range(_VREGS):
                    out.append(accs[u * _VREGS + k]
                               + bufb[2 * j + u, pl.ds(k * 16, 16)])
            return tuple(out)

        accs = tuple(jnp.zeros((16,), jnp.float32) for _ in range(2 * _VREGS))
        accs = lax.fori_loop(0, _LA // 2, jbody_a, accs)
        accs = lax.fori_loop(0, _LB // 2, jbody_b, accs)
        for k in range(_VREGS):
            pooled_v[r, pl.ds(k * 16, 16)] = accs[k] + accs[_VREGS + k]

    start(0, bufa0, bufb0, sem0)

    def pair_body(i, carry):
        r0 = 2 * i
        r1 = r0 + 1
        start(r1, bufa1, bufb1, sem1)
        wait(r0, bufa0, bufb0, sem0)
        rowsum(bufa0, bufb0, r0)

        @pl.when(i < _RPW // 2 - 1)
        def _():
            start(r0 + 2, bufa0, bufb0, sem0)

        wait(r1, bufa1, bufb1, sem1)
        rowsum(bufa1, bufb1, r1)
        return carry

    lax.fori_loop(0, _RPW // 2, pair_body, 0)
    pltpu.sync_copy(pooled_v, out_hbm.at[pl.ds(base, _RPW)])


@jax.jit
def _pool(xa, xb, table):
    mesh = plsc.VectorSubcoreMesh(core_axis_name="c", subcore_axis_name="s",
                                  num_cores=_NC)
    return pl.kernel(
        _pool_body,
        mesh=mesh,
        compiler_params=pltpu.CompilerParams(use_tc_tiling_on_sc=False),
        out_type=jax.ShapeDtypeStruct((_B, _D), jnp.float32),
        scratch_types=[
            pltpu.VMEM((_RPW, _LANES), jnp.int32),
            pltpu.VMEM((_RPW, _LANES), jnp.int32),
            pltpu.VMEM((_LA, _D), jnp.float32),
            pltpu.VMEM((_LB, _D), jnp.float32),
            pltpu.VMEM((_LA, _D), jnp.float32),
            pltpu.VMEM((_LB, _D), jnp.float32),
            pltpu.VMEM((_RPW, _D), jnp.float32),
            pltpu.SemaphoreType.DMA,
            pltpu.SemaphoreType.DMA,
        ],
    )(xa, xb, table)


def _head_body(pooled_ref, len_ref, w_ref, b_ref, o_ref):
    p = pooled_ref[...] / len_ref[...]
    logits = jnp.dot(p, w_ref[...], preferred_element_type=jnp.float32)
    logits = logits + b_ref[...]
    m = jnp.max(logits, axis=-1, keepdims=True)
    e = jnp.exp(logits - m)
    o_ref[...] = e / jnp.sum(e, axis=-1, keepdims=True)


@jax.jit
def _head(pooled, lens, w_pad, b_pad):
    return pl.pallas_call(
        _head_body,
        out_shape=jax.ShapeDtypeStruct((_B, _LANES), jnp.float32),
    )(pooled, lens, w_pad, b_pad)


def kernel(x, x_len, table, W, b):
    xi = x.astype(jnp.int32)
    xa = xi[:, :_LANES]
    xb = xi[:, _L - _LANES:_L]
    t_flat = lax.optimization_barrier(table.reshape(-1))
    pooled = _pool(xa, xb, t_flat.reshape(table.shape))
    lens = x_len.astype(jnp.float32).reshape(_B, 1)
    w_pad = jnp.pad(W, ((0, 0), (0, _LANES - _ASP)))
    b_pad = jnp.concatenate(
        [b, jnp.full((_LANES - _ASP,), -1e30, jnp.float32)]).reshape(1, _LANES)
    out = _head(pooled, lens, w_pad, b_pad)
    return out[:, :_ASP]
